# baseline (speedup 1.0000x reference)
import functools

import numpy as np
import jax
import jax.numpy as jnp
from jax import lax
from jax.experimental import pallas as pl
from jax.experimental.pallas import tpu as pltpu
from jax.experimental.pallas import tpu_sc as plsc

_B = 8
_M = 2048
_N = 33 * 33
_NPAD = 1104
_NC, _NS = 2, 16
_NW = _NC * _NS
_MW = _M // _NW
_NV = _MW // 16
_KCH = _NPAD // 16


def _interp_matrix():
    w = np.eye(5, dtype=np.float64)
    for _ in range(3):
        h = w.shape[0]
        out = np.zeros((2 * h - 1, w.shape[1]), dtype=np.float64)
        out[0::2] = w
        out[1::2] = 0.5 * (w[:-1] + w[1:])
        w = out
    return w


_WNP = _interp_matrix()
_KT_NP = np.kron(_WNP, _WNP).T
_KT = np.concatenate(
    [_KT_NP, np.zeros((25, _NPAD - _N))], axis=1
).astype(np.float32)


def _tc_coeffs(mf_ref, kt_ref, out_ref):
    xy = jnp.dot(mf_ref[...], kt_ref[...], preferred_element_type=jnp.float32)
    x = xy[0:8]
    y = xy[8:16]
    col = lax.broadcasted_iota(jnp.int32, (8, _NPAD), 1)
    pad = jnp.where(col >= _N, jnp.float32(1e30), jnp.float32(0.0))
    out_ref[0:8, :] = -2.0 * x
    out_ref[8:16, :] = -2.0 * y
    out_ref[16:24, :] = x * x + y * y + pad


_GATHER_DNUMS = lax.GatherDimensionNumbers(
    offset_dims=(), collapsed_slice_dims=(0,), start_index_map=(0,)
)


def _bcast(vec, j):
    idx = jnp.full((16, 1), j, dtype=jnp.int32)
    return lax.gather(
        vec,
        idx,
        _GATHER_DNUMS,
        slice_sizes=(1,),
        mode=lax.GatherScatterMode.PROMISE_IN_BOUNDS,
    )


@functools.cache
def _sc_chamfer_kernel():
    mesh = plsc.VectorSubcoreMesh(
        core_axis_name="c",
        subcore_axis_name="s",
        num_cores=_NC,
        num_subcores=_NS,
    )
    return pl.kernel(
        _sc_chamfer,
        mesh=mesh,
        out_type=jax.ShapeDtypeStruct((_NW, 16), jnp.float32),
        scratch_types=[
            pltpu.VMEM((24, _NPAD), jnp.float32),
            pltpu.VMEM((_B, 2, _MW), jnp.float32),
            pltpu.VMEM((16,), jnp.float32),
        ],
    )


def _sc_chamfer(abc_hbm, pcw_hbm, out_hbm, abc_v, pc_v, res_v):
    wid = lax.axis_index("c") * _NS + lax.axis_index("s")
    pltpu.sync_copy(abc_hbm, abc_v)
    pltpu.sync_copy(pcw_hbm.at[wid], pc_v)
    total = jnp.zeros((16,), jnp.float32)
    for b in range(_B):
        px = [pc_v[b, 0, pl.ds(v * 16, 16)] for v in range(_NV)]
        py = [pc_v[b, 1, pl.ds(v * 16, 16)] for v in range(_NV)]

        def body(k, accs, b=b, px=px, py=py):
            off = k * 16
            av = abc_v[b, pl.ds(off, 16)]
            bv = abc_v[8 + b, pl.ds(off, 16)]
            cv = abc_v[16 + b, pl.ds(off, 16)]
            new = list(accs)
            for j in range(16):
                aj = _bcast(av, j)
                bj = _bcast(bv, j)
                cj = _bcast(cv, j)
                for v in range(_NV):
                    t = aj * px[v] + bj * py[v] + cj
                    new[v] = jnp.minimum(new[v], t)
            return tuple(new)

        inf = jnp.full((16,), jnp.inf, jnp.float32)
        accs = lax.fori_loop(0, _KCH, body, (inf,) * _NV)
        for v in range(_NV):
            pp = px[v] * px[v] + py[v] * py[v]
            total = total + jnp.maximum(accs[v] + pp, 0.0)
    res_v[...] = total * jnp.float32(1.0 / (_B * _M))
    pltpu.sync_copy(res_v, out_hbm.at[wid])


@jax.jit
def kernel(network_mesh, pc):
    mx = network_mesh[:, 0].reshape(_B, 25)
    my = network_mesh[:, 1].reshape(_B, 25)
    mf = jnp.concatenate([mx, my], axis=0)
    abc = pl.pallas_call(
        _tc_coeffs,
        out_shape=jax.ShapeDtypeStruct((24, _NPAD), jnp.float32),
    )(mf, _KT)
    pcw = pc.reshape(_B, 2, _NW, _MW).transpose(2, 0, 1, 3)
    part = _sc_chamfer_kernel()(abc, pcw)
    return jnp.sum(part)
